# Initial kernel scaffold; baseline (speedup 1.0000x reference)
import jax
import jax.numpy as jnp
from jax.experimental import pallas as pl


def kernel(z, pos, batch, emb, W1, Wp, b1, W2, b2, std, mean):
    raise NotImplementedError("write your pallas kernel here")


# speedup vs baseline: 4.2210x; 4.2210x over previous
import jax
import jax.numpy as jnp
from jax import lax
from jax.experimental import pallas as pl
from jax.experimental.pallas import tpu as pltpu
from jax.experimental.pallas import tpu_sc as plsc

N = 500000
H = 128
NUM_Z = 100
M = 16384
NZP = 128

B = 2048
G = (N + B - 1) // B
NP = G * B

NW = 32
CHUNK = NP // NW
NVEC = CHUNK // 16


def _dense_body(z_ref, pos_ref, emb_ref, w1_ref, wp_ref, b1_ref, w2_ref,
                b2_ref, x_ref):
    g = pl.program_id(0)
    tab = jnp.dot(emb_ref[...], w1_ref[...],
                  preferred_element_type=jnp.float32) + b1_ref[...]
    zb = z_ref[0]
    ki = lax.broadcasted_iota(jnp.int32, (NZP, B), 0)
    oh = (ki == zb).astype(jnp.float32)
    h1 = lax.dot_general(tab, oh, (((0,), (0,)), ((), ())),
                         preferred_element_type=jnp.float32)
    pT = lax.dot_general(wp_ref[...], pos_ref[...], (((0,), (1,)), ((), ())),
                         preferred_element_type=jnp.float32)
    a = h1 + pT
    s = a * jax.nn.sigmoid(a)
    xT = lax.dot_general(w2_ref[...], s, (((1,), (0,)), ((), ())),
                         preferred_element_type=jnp.float32)
    xT = xT + b2_ref[0, 0]
    idx = lax.broadcasted_iota(jnp.int32, (1, B), 1) + g * B
    x_ref[...] = jnp.where(idx < N, xT, 0.0).reshape(1, 1, B)


def _dense_stage(z3, pos, embp, W1, Wp, b1r, w2r, b2s):
    return pl.pallas_call(
        _dense_body,
        grid=(G,),
        in_specs=[
            pl.BlockSpec((1, 1, B), lambda g: (g, 0, 0)),
            pl.BlockSpec((B, 3), lambda g: (g, 0)),
            pl.BlockSpec((NZP, H), lambda g: (0, 0)),
            pl.BlockSpec((H, H), lambda g: (0, 0)),
            pl.BlockSpec((3, H), lambda g: (0, 0)),
            pl.BlockSpec((1, H), lambda g: (0, 0)),
            pl.BlockSpec((1, H), lambda g: (0, 0)),
            pl.BlockSpec(memory_space=pltpu.SMEM),
        ],
        out_specs=pl.BlockSpec((1, 1, B), lambda g: (g, 0, 0)),
        out_shape=jax.ShapeDtypeStruct((G, 1, B), jnp.float32),
    )(z3, pos, embp, W1, Wp, b1r, w2r, b2s)


def _seg_body(x_hbm, b_hbm, bn_hbm, bp_hbm, out_hbm, xv, bv, bnv, bpv, acc,
              tmp):
    wid = lax.axis_index("s") * 2 + lax.axis_index("c")
    base = wid * CHUNK
    pltpu.sync_copy(x_hbm.at[pl.ds(base, CHUNK)], xv)
    pltpu.sync_copy(b_hbm.at[pl.ds(base, CHUNK)], bv)
    pltpu.sync_copy(bn_hbm.at[pl.ds(base, CHUNK)], bnv)
    pltpu.sync_copy(bp_hbm.at[pl.ds(base, CHUNK)], bpv)

    zeros16 = jnp.zeros((16,), jnp.float32)

    def zero_body(i, carry):
        acc[pl.ds(i * 16, 16)] = zeros16
        return carry

    lax.fori_loop(0, M // 16, zero_body, 0)

    ii = lax.broadcasted_iota(jnp.int32, (16,), 0)
    lane0 = ii == 0
    lane15 = ii == 15

    def body(i, carry):
        o = i * 16
        xvec = xv[pl.ds(o, 16)]
        b = bv[pl.ds(o, 16)]
        bn = bnv[pl.ds(o, 16)]
        bp = bpv[pl.ds(o, 16)]
        c = plsc.cumsum(xvec)
        excl = c - xvec
        last = (b != bn) | lane15
        first = (b != bp) | lane0
        plsc.store_compressed(tmp.at[...], excl, mask=first)
        y = plsc.load_expanded(tmp.at[...], mask=last)
        d = c - y
        plsc.addupdate_scatter(acc, [b], d, mask=last)
        return carry

    lax.fori_loop(0, NVEC, body, 0)
    pltpu.sync_copy(acc, out_hbm.at[wid])


def _segment_stage(x_flat, batch_pad, bnext, bprev):
    mesh = plsc.VectorSubcoreMesh(core_axis_name="c", subcore_axis_name="s")
    fn = pl.kernel(
        _seg_body,
        out_type=jax.ShapeDtypeStruct((NW, M), jnp.float32),
        mesh=mesh,
        compiler_params=pltpu.CompilerParams(needs_layout_passes=False),
        scratch_types=[
            pltpu.VMEM((CHUNK,), jnp.float32),
            pltpu.VMEM((CHUNK,), jnp.int32),
            pltpu.VMEM((CHUNK,), jnp.int32),
            pltpu.VMEM((CHUNK,), jnp.int32),
            pltpu.VMEM((M,), jnp.float32),
            pltpu.VMEM((16,), jnp.float32),
        ],
    )
    return fn(x_flat, batch_pad, bnext, bprev)


def _combine_body(p_ref, std_ref, mean_ref, o_ref):
    o_ref[...] = (jnp.sum(p_ref[...], axis=0, keepdims=True)
                  * std_ref[0, 0] + mean_ref[0, 0])


def _combine_stage(partial, std2, mean2):
    return pl.pallas_call(
        _combine_body,
        in_specs=[
            pl.BlockSpec((NW, M), lambda: (0, 0)),
            pl.BlockSpec(memory_space=pltpu.SMEM),
            pl.BlockSpec(memory_space=pltpu.SMEM),
        ],
        out_specs=pl.BlockSpec((1, M), lambda: (0, 0)),
        out_shape=jax.ShapeDtypeStruct((1, M), jnp.float32),
    )(partial, std2, mean2)


def kernel(z, pos, batch, emb, W1, Wp, b1, W2, b2, std, mean):
    z = z.astype(jnp.int32)
    batch = batch.astype(jnp.int32)

    z3 = jnp.concatenate([z, jnp.zeros((NP - N,), jnp.int32)]).reshape(G, 1, B)
    embp = jnp.concatenate(
        [emb, jnp.zeros((NZP - NUM_Z, H), jnp.float32)], axis=0)
    b1r = b1.reshape(1, H)
    w2r = W2.reshape(1, H)
    b2s = b2.reshape(1, 1)

    x3 = _dense_stage(z3, pos, embp, W1, Wp, b1r, w2r, b2s)
    x_flat = x3.reshape(NP)

    batch_pad = jnp.concatenate(
        [batch, jnp.full((NP - N,), M - 1, jnp.int32)])
    bnext = jnp.concatenate([batch_pad[1:], jnp.full((1,), M, jnp.int32)])
    bprev = jnp.concatenate([jnp.full((1,), -1, jnp.int32), batch_pad[:-1]])

    partial = _segment_stage(x_flat, batch_pad, bnext, bprev)

    out = _combine_stage(partial, std.reshape(1, 1), mean.reshape(1, 1))
    return out.reshape(M, 1)


# speedup vs baseline: 7.1204x; 1.6869x over previous
import jax
import jax.numpy as jnp
from jax import lax
from jax.experimental import pallas as pl
from jax.experimental.pallas import tpu as pltpu
from jax.experimental.pallas import tpu_sc as plsc

N = 500000
H = 128
NUM_Z = 100
M = 16384
NZP = 128

B = 4096
G = (N + B - 1) // B
NP = G * B

NW = 32
CHUNK = NP // NW
NVEC = CHUNK // 16


def _dense_body(z_ref, pos_ref, emb_ref, w1_ref, wp_ref, b1_ref, w2_ref,
                b2_ref, x_ref):
    g = pl.program_id(0)
    tab = jnp.dot(emb_ref[...], w1_ref[...],
                  preferred_element_type=jnp.float32) + b1_ref[...]
    zb = z_ref[0].astype(jnp.int16)
    ki = lax.broadcasted_iota(jnp.int16, (NZP, B), 0)
    oh = jnp.where(ki == zb, jnp.bfloat16(1), jnp.bfloat16(0))
    tab_hi = tab.astype(jnp.bfloat16)
    tab_lo = (tab - tab_hi.astype(jnp.float32)).astype(jnp.bfloat16)
    dn = (((0,), (0,)), ((), ()))
    h1 = (lax.dot_general(tab_hi, oh, dn, preferred_element_type=jnp.float32)
          + lax.dot_general(tab_lo, oh, dn,
                            preferred_element_type=jnp.float32))
    pT = lax.dot_general(wp_ref[...], pos_ref[...], (((0,), (0,)), ((), ())),
                         preferred_element_type=jnp.float32)
    a = h1 + pT
    s = a * (0.5 * jnp.tanh(a * 0.5) + 0.5)
    xT = lax.dot_general(w2_ref[...], s, (((1,), (0,)), ((), ())),
                         preferred_element_type=jnp.float32)
    xT = xT + b2_ref[0, 0]
    idx = lax.broadcasted_iota(jnp.int32, (1, B), 1) + g * B
    x_ref[...] = jnp.where(idx < N, xT, 0.0).reshape(1, 1, B)


def _dense_stage(z3, posT, embp, W1, Wp, b1r, w2r, b2s):
    return pl.pallas_call(
        _dense_body,
        grid=(G,),
        in_specs=[
            pl.BlockSpec((1, 1, B), lambda g: (g, 0, 0)),
            pl.BlockSpec((3, B), lambda g: (0, g)),
            pl.BlockSpec((NZP, H), lambda g: (0, 0)),
            pl.BlockSpec((H, H), lambda g: (0, 0)),
            pl.BlockSpec((3, H), lambda g: (0, 0)),
            pl.BlockSpec((1, H), lambda g: (0, 0)),
            pl.BlockSpec((1, H), lambda g: (0, 0)),
            pl.BlockSpec(memory_space=pltpu.SMEM),
        ],
        out_specs=pl.BlockSpec((1, 1, B), lambda g: (g, 0, 0)),
        out_shape=jax.ShapeDtypeStruct((G, 1, B), jnp.float32),
    )(z3, posT, embp, W1, Wp, b1r, w2r, b2s)


def _seg_body(x_hbm, b_hbm, bn_hbm, bp_hbm, out_hbm, xv, bv, bnv, bpv, acc,
              tmp):
    wid = lax.axis_index("s") * 2 + lax.axis_index("c")
    base = wid * CHUNK
    pltpu.sync_copy(x_hbm.at[pl.ds(base, CHUNK)], xv)
    pltpu.sync_copy(b_hbm.at[pl.ds(base, CHUNK)], bv)
    pltpu.sync_copy(bn_hbm.at[pl.ds(base, CHUNK)], bnv)
    pltpu.sync_copy(bp_hbm.at[pl.ds(base, CHUNK)], bpv)

    zeros16 = jnp.zeros((16,), jnp.float32)

    def zero_body(i, carry):
        acc[pl.ds(i * 16, 16)] = zeros16
        return carry

    lax.fori_loop(0, M // 16, zero_body, 0)

    ii = lax.broadcasted_iota(jnp.int32, (16,), 0)
    lane0 = ii == 0
    lane15 = ii == 15

    def body(i, carry):
        o = i * 16
        xvec = xv[pl.ds(o, 16)]
        b = bv[pl.ds(o, 16)]
        bn = bnv[pl.ds(o, 16)]
        bp = bpv[pl.ds(o, 16)]
        c = plsc.cumsum(xvec)
        excl = c - xvec
        last = (b != bn) | lane15
        first = (b != bp) | lane0
        plsc.store_compressed(tmp.at[...], excl, mask=first)
        y = plsc.load_expanded(tmp.at[...], mask=last)
        d = c - y
        plsc.addupdate_scatter(acc, [b], d, mask=last)
        return carry

    lax.fori_loop(0, NVEC, body, 0)
    pltpu.sync_copy(acc, out_hbm.at[wid])


def _segment_stage(x_flat, batch_pad, bnext, bprev):
    mesh = plsc.VectorSubcoreMesh(core_axis_name="c", subcore_axis_name="s")
    fn = pl.kernel(
        _seg_body,
        out_type=jax.ShapeDtypeStruct((NW, M), jnp.float32),
        mesh=mesh,
        compiler_params=pltpu.CompilerParams(needs_layout_passes=False),
        scratch_types=[
            pltpu.VMEM((CHUNK,), jnp.float32),
            pltpu.VMEM((CHUNK,), jnp.int32),
            pltpu.VMEM((CHUNK,), jnp.int32),
            pltpu.VMEM((CHUNK,), jnp.int32),
            pltpu.VMEM((M,), jnp.float32),
            pltpu.VMEM((16,), jnp.float32),
        ],
    )
    return fn(x_flat, batch_pad, bnext, bprev)


def _combine_body(p_ref, std_ref, mean_ref, o_ref):
    o_ref[...] = (jnp.sum(p_ref[...], axis=0, keepdims=True)
                  * std_ref[0, 0] + mean_ref[0, 0])


def _combine_stage(partial, std2, mean2):
    return pl.pallas_call(
        _combine_body,
        in_specs=[
            pl.BlockSpec((NW, M), lambda: (0, 0)),
            pl.BlockSpec(memory_space=pltpu.SMEM),
            pl.BlockSpec(memory_space=pltpu.SMEM),
        ],
        out_specs=pl.BlockSpec((1, M), lambda: (0, 0)),
        out_shape=jax.ShapeDtypeStruct((1, M), jnp.float32),
    )(partial, std2, mean2)


def kernel(z, pos, batch, emb, W1, Wp, b1, W2, b2, std, mean):
    z = z.astype(jnp.int32)
    batch = batch.astype(jnp.int32)

    z3 = jnp.concatenate([z, jnp.zeros((NP - N,), jnp.int32)]).reshape(G, 1, B)
    posT = jnp.pad(pos.T, ((0, 0), (0, NP - N)))
    embp = jnp.concatenate(
        [emb, jnp.zeros((NZP - NUM_Z, H), jnp.float32)], axis=0)
    b1r = b1.reshape(1, H)
    w2r = W2.reshape(1, H)
    b2s = b2.reshape(1, 1)

    x3 = _dense_stage(z3, posT, embp, W1, Wp, b1r, w2r, b2s)
    x_flat = x3.reshape(NP)

    batch_pad = jnp.concatenate(
        [batch, jnp.full((NP - N,), M - 1, jnp.int32)])
    bnext = jnp.concatenate([batch_pad[1:], jnp.full((1,), M, jnp.int32)])
    bprev = jnp.concatenate([jnp.full((1,), -1, jnp.int32), batch_pad[:-1]])

    partial = _segment_stage(x_flat, batch_pad, bnext, bprev)

    out = _combine_stage(partial, std.reshape(1, 1), mean.reshape(1, 1))
    return out.reshape(M, 1)


# speedup vs baseline: 7.4060x; 1.0401x over previous
import jax
import jax.numpy as jnp
from jax import lax
from jax.experimental import pallas as pl
from jax.experimental.pallas import tpu as pltpu
from jax.experimental.pallas import tpu_sc as plsc

N = 500000
H = 128
NUM_Z = 100
M = 16384
NZP = 128

B = 4096
G = (N + B - 1) // B
NP = G * B

NW = 32
CHUNK = NP // NW
NVEC = CHUNK // 16


def _dense_body(z_ref, pos_ref, emb_ref, w1_ref, wp_ref, b1_ref, w2_ref,
                b2_ref, x_ref):
    g = pl.program_id(0)
    tab = jnp.dot(emb_ref[...], w1_ref[...],
                  preferred_element_type=jnp.float32) + b1_ref[...]
    zb = z_ref[0].astype(jnp.int16)
    ki = lax.broadcasted_iota(jnp.int16, (NZP, B), 0)
    oh = jnp.where(ki == zb, jnp.bfloat16(1), jnp.bfloat16(0))
    tab_hi = tab.astype(jnp.bfloat16)
    tab_lo = (tab - tab_hi.astype(jnp.float32)).astype(jnp.bfloat16)
    dn = (((0,), (0,)), ((), ()))
    h1 = (lax.dot_general(tab_hi, oh, dn, preferred_element_type=jnp.float32)
          + lax.dot_general(tab_lo, oh, dn,
                            preferred_element_type=jnp.float32))
    pT = lax.dot_general(wp_ref[...], pos_ref[...], (((0,), (0,)), ((), ())),
                         preferred_element_type=jnp.float32)
    a = h1 + pT
    s = a * (0.5 * jnp.tanh(a * 0.5) + 0.5)
    xT = lax.dot_general(w2_ref[...], s, (((1,), (0,)), ((), ())),
                         preferred_element_type=jnp.float32)
    xT = xT + b2_ref[0, 0]
    idx = lax.broadcasted_iota(jnp.int32, (1, B), 1) + g * B
    x_ref[...] = jnp.where(idx < N, xT, 0.0).reshape(1, 1, B)


def _dense_stage(z3, posT, embp, W1, Wp, b1r, w2r, b2s):
    return pl.pallas_call(
        _dense_body,
        grid=(G,),
        in_specs=[
            pl.BlockSpec((1, 1, B), lambda g: (g, 0, 0)),
            pl.BlockSpec((3, B), lambda g: (0, g)),
            pl.BlockSpec((NZP, H), lambda g: (0, 0)),
            pl.BlockSpec((H, H), lambda g: (0, 0)),
            pl.BlockSpec((3, H), lambda g: (0, 0)),
            pl.BlockSpec((1, H), lambda g: (0, 0)),
            pl.BlockSpec((1, H), lambda g: (0, 0)),
            pl.BlockSpec(memory_space=pltpu.SMEM),
        ],
        out_specs=pl.BlockSpec((1, 1, B), lambda g: (g, 0, 0)),
        out_shape=jax.ShapeDtypeStruct((G, 1, B), jnp.float32),
    )(z3, posT, embp, W1, Wp, b1r, w2r, b2s)


def _seg_body(x_hbm, b_hbm, bn_hbm, bp_hbm, out_hbm, xv, bv, bnv, bpv, acc,
              tmp, sem):
    wid = lax.axis_index("s") * 2 + lax.axis_index("c")
    base = wid * CHUNK
    cp1 = pltpu.async_copy(x_hbm.at[pl.ds(base, CHUNK)], xv, sem)
    cp2 = pltpu.async_copy(b_hbm.at[pl.ds(base, CHUNK)], bv, sem)
    cp3 = pltpu.async_copy(bn_hbm.at[pl.ds(base, CHUNK)], bnv, sem)
    cp4 = pltpu.async_copy(bp_hbm.at[pl.ds(base, CHUNK)], bpv, sem)

    zeros16 = jnp.zeros((16,), jnp.float32)

    def zero_body(i, carry):
        acc[pl.ds(i * 16, 16)] = zeros16
        return carry

    lax.fori_loop(0, M // 16, zero_body, 0, unroll=8)
    cp1.wait()
    cp2.wait()
    cp3.wait()
    cp4.wait()

    ii = lax.broadcasted_iota(jnp.int32, (16,), 0)
    lane0 = ii == 0
    lane15 = ii == 15

    def body(i, carry):
        o = i * 16
        xvec = xv[pl.ds(o, 16)]
        b = bv[pl.ds(o, 16)]
        bn = bnv[pl.ds(o, 16)]
        bp = bpv[pl.ds(o, 16)]
        c = plsc.cumsum(xvec)
        excl = c - xvec
        last = (b != bn) | lane15
        first = (b != bp) | lane0
        plsc.store_compressed(tmp.at[...], excl, mask=first)
        y = plsc.load_expanded(tmp.at[...], mask=last)
        d = c - y
        plsc.addupdate_scatter(acc, [b], d, mask=last)
        return carry

    lax.fori_loop(0, NVEC, body, 0, unroll=4)
    pltpu.sync_copy(acc, out_hbm.at[wid])


def _segment_stage(x_flat, batch_pad, bnext, bprev):
    mesh = plsc.VectorSubcoreMesh(core_axis_name="c", subcore_axis_name="s")
    fn = pl.kernel(
        _seg_body,
        out_type=jax.ShapeDtypeStruct((NW, M), jnp.float32),
        mesh=mesh,
        compiler_params=pltpu.CompilerParams(needs_layout_passes=False),
        scratch_types=[
            pltpu.VMEM((CHUNK,), jnp.float32),
            pltpu.VMEM((CHUNK,), jnp.int32),
            pltpu.VMEM((CHUNK,), jnp.int32),
            pltpu.VMEM((CHUNK,), jnp.int32),
            pltpu.VMEM((M,), jnp.float32),
            pltpu.VMEM((16,), jnp.float32),
            pltpu.SemaphoreType.DMA,
        ],
    )
    return fn(x_flat, batch_pad, bnext, bprev)


def _combine_body(p_ref, std_ref, mean_ref, o_ref):
    o_ref[...] = (jnp.sum(p_ref[...], axis=0, keepdims=True)
                  * std_ref[0, 0] + mean_ref[0, 0])


def _combine_stage(partial, std2, mean2):
    return pl.pallas_call(
        _combine_body,
        in_specs=[
            pl.BlockSpec((NW, M), lambda: (0, 0)),
            pl.BlockSpec(memory_space=pltpu.SMEM),
            pl.BlockSpec(memory_space=pltpu.SMEM),
        ],
        out_specs=pl.BlockSpec((1, M), lambda: (0, 0)),
        out_shape=jax.ShapeDtypeStruct((1, M), jnp.float32),
    )(partial, std2, mean2)


def kernel(z, pos, batch, emb, W1, Wp, b1, W2, b2, std, mean):
    z = z.astype(jnp.int32)
    batch = batch.astype(jnp.int32)

    z3 = jnp.concatenate([z, jnp.zeros((NP - N,), jnp.int32)]).reshape(G, 1, B)
    posT = jnp.pad(pos.T, ((0, 0), (0, NP - N)))
    embp = jnp.concatenate(
        [emb, jnp.zeros((NZP - NUM_Z, H), jnp.float32)], axis=0)
    b1r = b1.reshape(1, H)
    w2r = W2.reshape(1, H)
    b2s = b2.reshape(1, 1)

    x3 = _dense_stage(z3, posT, embp, W1, Wp, b1r, w2r, b2s)
    x_flat = x3.reshape(NP)

    batch_pad = jnp.concatenate(
        [batch, jnp.full((NP - N,), M - 1, jnp.int32)])
    bnext = jnp.concatenate([batch_pad[1:], jnp.full((1,), M, jnp.int32)])
    bprev = jnp.concatenate([jnp.full((1,), -1, jnp.int32), batch_pad[:-1]])

    partial = _segment_stage(x_flat, batch_pad, bnext, bprev)

    out = _combine_stage(partial, std.reshape(1, 1), mean.reshape(1, 1))
    return out.reshape(M, 1)


# speedup vs baseline: 10.6081x; 1.4324x over previous
import jax
import jax.numpy as jnp
from jax import lax
from jax.experimental import pallas as pl
from jax.experimental.pallas import tpu as pltpu
from jax.experimental.pallas import tpu_sc as plsc

N = 500000
H = 128
NUM_Z = 100
M = 16384
NZP = 128

B = 8192
G = (N + B - 1) // B
NP = G * B

NW = 32
CHUNK = NP // NW
NVEC = CHUNK // 16


def _dense_body(z_ref, pos_ref, emb_ref, w1_ref, wp_ref, b1_ref, w2_ref,
                b2_ref, x_ref):
    g = pl.program_id(0)
    tab = jnp.dot(emb_ref[...], w1_ref[...],
                  preferred_element_type=jnp.float32) + b1_ref[...]
    zb = z_ref[0].astype(jnp.int16)
    ki = lax.broadcasted_iota(jnp.int16, (NZP, B), 0)
    oh = jnp.where(ki == zb, jnp.bfloat16(1), jnp.bfloat16(0))
    tab_hi = tab.astype(jnp.bfloat16)
    tab_lo = (tab - tab_hi.astype(jnp.float32)).astype(jnp.bfloat16)
    wp = wp_ref[...]
    wp_hi = wp.astype(jnp.bfloat16)
    wp_lo = (wp - wp_hi.astype(jnp.float32)).astype(jnp.bfloat16)
    po = pos_ref[...]
    po_hi = po.astype(jnp.bfloat16)
    po_lo = (po - po_hi.astype(jnp.float32)).astype(jnp.bfloat16)
    lhs = jnp.concatenate([tab_hi, tab_lo, wp_hi, wp_hi, wp_lo], axis=0)
    rhs = jnp.concatenate([oh, oh, po_hi, po_lo, po_hi], axis=0)
    a = lax.dot_general(lhs, rhs, (((0,), (0,)), ((), ())),
                        preferred_element_type=jnp.float32)
    s = a * (0.5 * jnp.tanh(a * 0.5) + 0.5)
    xT = lax.dot_general(w2_ref[...], s, (((1,), (0,)), ((), ())),
                         preferred_element_type=jnp.float32)
    xT = xT + b2_ref[0, 0]
    idx = lax.broadcasted_iota(jnp.int32, (1, B), 1) + g * B
    x_ref[...] = jnp.where(idx < N, xT, 0.0).reshape(1, 1, B)


def _dense_stage(z3, posT, embp, W1, Wp, b1r, w2r, b2s):
    return pl.pallas_call(
        _dense_body,
        grid=(G,),
        in_specs=[
            pl.BlockSpec((1, 1, B), lambda g: (g, 0, 0)),
            pl.BlockSpec((3, B), lambda g: (0, g)),
            pl.BlockSpec((NZP, H), lambda g: (0, 0)),
            pl.BlockSpec((H, H), lambda g: (0, 0)),
            pl.BlockSpec((3, H), lambda g: (0, 0)),
            pl.BlockSpec((1, H), lambda g: (0, 0)),
            pl.BlockSpec((1, H), lambda g: (0, 0)),
            pl.BlockSpec(memory_space=pltpu.SMEM),
        ],
        out_specs=pl.BlockSpec((1, 1, B), lambda g: (g, 0, 0)),
        out_shape=jax.ShapeDtypeStruct((G, 1, B), jnp.float32),
    )(z3, posT, embp, W1, Wp, b1r, w2r, b2s)


def _seg_body(x_hbm, b_hbm, bn_hbm, bp_hbm, out_hbm, xv, bv, bnv, bpv, acc,
              tmp, sem):
    wid = lax.axis_index("s") * 2 + lax.axis_index("c")
    base = wid * CHUNK
    cp1 = pltpu.async_copy(x_hbm.at[pl.ds(base, CHUNK)], xv, sem)
    cp2 = pltpu.async_copy(b_hbm.at[pl.ds(base, CHUNK)], bv, sem)
    cp3 = pltpu.async_copy(bn_hbm.at[pl.ds(base, CHUNK)], bnv, sem)
    cp4 = pltpu.async_copy(bp_hbm.at[pl.ds(base, CHUNK)], bpv, sem)

    zeros16 = jnp.zeros((16,), jnp.float32)

    def zero_body(i, carry):
        acc[pl.ds(i * 16, 16)] = zeros16
        return carry

    lax.fori_loop(0, M // 16, zero_body, 0, unroll=8)
    cp1.wait()
    cp2.wait()
    cp3.wait()
    cp4.wait()

    ii = lax.broadcasted_iota(jnp.int32, (16,), 0)
    lane0 = ii == 0
    lane15 = ii == 15

    def body(i, carry):
        o = i * 16
        xvec = xv[pl.ds(o, 16)]
        b = bv[pl.ds(o, 16)]
        bn = bnv[pl.ds(o, 16)]
        bp = bpv[pl.ds(o, 16)]
        c = plsc.cumsum(xvec)
        excl = c - xvec
        last = (b != bn) | lane15
        first = (b != bp) | lane0
        plsc.store_compressed(tmp.at[...], excl, mask=first)
        y = plsc.load_expanded(tmp.at[...], mask=last)
        d = c - y
        plsc.addupdate_scatter(acc, [b], d, mask=last)
        return carry

    lax.fori_loop(0, NVEC, body, 0, unroll=4)
    pltpu.sync_copy(acc, out_hbm.at[wid])


def _segment_stage(x_flat, batch_pad, bnext, bprev):
    mesh = plsc.VectorSubcoreMesh(core_axis_name="c", subcore_axis_name="s")
    fn = pl.kernel(
        _seg_body,
        out_type=jax.ShapeDtypeStruct((NW, M), jnp.float32),
        mesh=mesh,
        compiler_params=pltpu.CompilerParams(needs_layout_passes=False),
        scratch_types=[
            pltpu.VMEM((CHUNK,), jnp.float32),
            pltpu.VMEM((CHUNK,), jnp.int32),
            pltpu.VMEM((CHUNK,), jnp.int32),
            pltpu.VMEM((CHUNK,), jnp.int32),
            pltpu.VMEM((M,), jnp.float32),
            pltpu.VMEM((16,), jnp.float32),
            pltpu.SemaphoreType.DMA,
        ],
    )
    return fn(x_flat, batch_pad, bnext, bprev)


def _combine_body(p_ref, std_ref, mean_ref, o_ref):
    o_ref[...] = (jnp.sum(p_ref[...], axis=0, keepdims=True)
                  * std_ref[0, 0] + mean_ref[0, 0])


def _combine_stage(partial, std2, mean2):
    return pl.pallas_call(
        _combine_body,
        in_specs=[
            pl.BlockSpec((NW, M), lambda: (0, 0)),
            pl.BlockSpec(memory_space=pltpu.SMEM),
            pl.BlockSpec(memory_space=pltpu.SMEM),
        ],
        out_specs=pl.BlockSpec((1, M), lambda: (0, 0)),
        out_shape=jax.ShapeDtypeStruct((1, M), jnp.float32),
    )(partial, std2, mean2)


def kernel(z, pos, batch, emb, W1, Wp, b1, W2, b2, std, mean):
    z = z.astype(jnp.int32)
    batch = batch.astype(jnp.int32)

    z3 = jnp.concatenate([z, jnp.zeros((NP - N,), jnp.int32)]).reshape(G, 1, B)
    posT = jnp.pad(pos.T, ((0, 0), (0, NP - N)))
    embp = jnp.concatenate(
        [emb, jnp.zeros((NZP - NUM_Z, H), jnp.float32)], axis=0)
    b1r = b1.reshape(1, H)
    w2r = W2.reshape(1, H)
    b2s = b2.reshape(1, 1)

    x3 = _dense_stage(z3, posT, embp, W1, Wp, b1r, w2r, b2s)
    x_flat = x3.reshape(NP)

    batch_pad = jnp.concatenate(
        [batch, jnp.full((NP - N,), M - 1, jnp.int32)])
    bnext = jnp.concatenate([batch_pad[1:], jnp.full((1,), M, jnp.int32)])
    bprev = jnp.concatenate([jnp.full((1,), -1, jnp.int32), batch_pad[:-1]])

    partial = _segment_stage(x_flat, batch_pad, bnext, bprev)

    out = _combine_stage(partial, std.reshape(1, 1), mean.reshape(1, 1))
    return out.reshape(M, 1)
